# baseline (speedup 1.0000x reference)
import functools

import jax
import jax.numpy as jnp
from jax import lax
from jax.experimental import pallas as pl
from jax.experimental.pallas import tpu as pltpu
from jax.experimental.pallas import tpu_sc as plsc

_N = 10000
_E = 160000
_NHEADS = 4
_ALPHA = 0.01

_NC = 2
_NS = 16
_NW = _NC * _NS
_L = 16
_EPW = 5008
_EPAD = _NW * _EPW
_NPAD = 10240
_COLS = _NPAD // _NS
_CH = 512
_NCHUNK = _NPAD // _CH
_WHROWS = _N + 8
_NIT = _EPW // _L
_G = 16

_mesh = plsc.VectorSubcoreMesh(core_axis_name="c", subcore_axis_name="s")


def _mm_scores_body(x_ref, w_ref, a_ref, wh_ref, s_ref, *, elu_input):
    x = x_ref[...]
    if elu_input:
        x = jnp.where(x > 0, x, jnp.exp(x) - 1.0)
    wh = jnp.dot(x, w_ref[...], preferred_element_type=jnp.float32)
    wh_ref[...] = wh
    s_ref[...] = jnp.dot(wh, a_ref[...], preferred_element_type=jnp.float32)


def _mm_scores(x, W, A, block_rows=1000, elu_input=False):
    n, k = x.shape
    m = W.shape[1]
    c = A.shape[1]
    return pl.pallas_call(
        functools.partial(_mm_scores_body, elu_input=elu_input),
        grid=(n // block_rows,),
        in_specs=[
            pl.BlockSpec((block_rows, k), lambda i: (i, 0)),
            pl.BlockSpec((k, m), lambda i: (0, 0)),
            pl.BlockSpec((m, c), lambda i: (0, 0)),
        ],
        out_specs=[
            pl.BlockSpec((block_rows, m), lambda i: (i, 0)),
            pl.BlockSpec((block_rows, c), lambda i: (i, 0)),
        ],
        out_shape=[
            jax.ShapeDtypeStruct((n, m), jnp.float32),
            jax.ShapeDtypeStruct((n, c), jnp.float32),
        ],
    )(x, W, A)


def _mlp_body(emb_ref, w1_ref, b1_ref, g_ref, b_ref, w2_ref, b2_ref, out_ref):
    g = jnp.dot(emb_ref[...], w1_ref[...], preferred_element_type=jnp.float32)
    g = g + b1_ref[...]
    g = jnp.maximum(g, 0.0)
    mu = jnp.mean(g, axis=-1, keepdims=True)
    var = jnp.mean((g - mu) ** 2, axis=-1, keepdims=True)
    g = (g - mu) / jnp.sqrt(var + 1e-5) * g_ref[...] + b_ref[...]
    out_ref[...] = jnp.dot(g, w2_ref[...], preferred_element_type=jnp.float32) + b2_ref[...]


def _mlp(emb, w1, b1, ln_g, ln_b, w2, b2, block_rows=1000):
    n, k = emb.shape
    h = w1.shape[1]
    m = w2.shape[1]
    return pl.pallas_call(
        _mlp_body,
        grid=(n // block_rows,),
        in_specs=[
            pl.BlockSpec((block_rows, k), lambda i: (i, 0)),
            pl.BlockSpec((k, h), lambda i: (0, 0)),
            pl.BlockSpec((1, h), lambda i: (0, 0)),
            pl.BlockSpec((1, h), lambda i: (0, 0)),
            pl.BlockSpec((1, h), lambda i: (0, 0)),
            pl.BlockSpec((h, m), lambda i: (0, 0)),
            pl.BlockSpec((1, m), lambda i: (0, 0)),
        ],
        out_specs=pl.BlockSpec((block_rows, m), lambda i: (i, 0)),
        out_shape=jax.ShapeDtypeStruct((n, m), jnp.float32),
    )(emb, w1, b1.reshape(1, h), ln_g.reshape(1, h),
      ln_b.reshape(1, h), w2, b2.reshape(1, m))


def _att_a_body(H, sd_hbm, src_hbm, dst_hbm, score_hbm, maxpart_hbm,
                sd_v, src_v, dst_v, score_v, mx_v, red_v, shared_mx):
    c = lax.axis_index("c")
    t = lax.axis_index("s")
    w = c * _NS + t
    pltpu.sync_copy(sd_hbm, sd_v)
    pltpu.sync_copy(src_hbm.at[pl.ds(w * _EPW, _EPW)], src_v)
    pltpu.sync_copy(dst_hbm.at[pl.ds(w * _EPW, _EPW)], dst_v)

    def body(i, carry):
        b = i * _L
        sv = src_v[pl.ds(b, _L)]
        dv = dst_v[pl.ds(b, _L)]
        out = []
        for h in range(H):
            e = (plsc.load_gather(sd_v, [sv + h * _NPAD])
                 + plsc.load_gather(sd_v, [dv + (H + h) * _NPAD]))
            e = jnp.where(e > 0, e, _ALPHA * e)
            score_v[pl.ds(h * _EPW + b, _L)] = e
            out.append(jnp.maximum(carry[h], e))
        return tuple(out)

    init = tuple(jnp.full((_L,), -1e30, jnp.float32) for _ in range(H))
    mx = lax.fori_loop(0, _NIT, body, init)
    for h in range(H):
        mx_v[h, :] = mx[h]
    pltpu.sync_copy(score_v, score_hbm.at[w])
    pltpu.sync_copy(mx_v, shared_mx.at[t])
    plsc.subcore_barrier()

    @pl.when(t == 0)
    def _():
        pltpu.sync_copy(shared_mx, red_v)
        for h in range(H):
            m = red_v[0, h, :]
            for tt in range(1, _NS):
                m = jnp.maximum(m, red_v[tt, h, :])
            mx_v[h, :] = m
        pltpu.sync_copy(mx_v, maxpart_hbm.at[c])


def _att_a(H, sd, src, dst):
    k = pl.kernel(
        functools.partial(_att_a_body, H),
        out_type=[
            jax.ShapeDtypeStruct((_NW, H * _EPW), jnp.float32),
            jax.ShapeDtypeStruct((_NC, H, _L), jnp.float32),
        ],
        mesh=_mesh,
        compiler_params=pltpu.CompilerParams(needs_layout_passes=False),
        scratch_types=[
            pltpu.VMEM((2 * H * _NPAD,), jnp.float32),
            pltpu.VMEM((_EPW,), jnp.int32),
            pltpu.VMEM((_EPW,), jnp.int32),
            pltpu.VMEM((H * _EPW,), jnp.float32),
            pltpu.VMEM((H, _L), jnp.float32),
            pltpu.VMEM((_NS, H, _L), jnp.float32),
            pltpu.VMEM_SHARED((_NS, H, _L), jnp.float32),
        ],
    )
    return k(sd, src, dst)


def _att_b_body(H, score_hbm, dst_hbm, maxpart_hbm, att_hbm, denpart_hbm,
                score_v, dst_v, mxp_v, den_v):
    c = lax.axis_index("c")
    t = lax.axis_index("s")
    w = c * _NS + t
    pltpu.sync_copy(score_hbm.at[w], score_v)
    pltpu.sync_copy(dst_hbm.at[pl.ds(w * _EPW, _EPW)], dst_v)
    pltpu.sync_copy(maxpart_hbm, mxp_v)

    zf = jnp.zeros((_L,), jnp.float32)

    def zbody(i, _):
        den_v[pl.ds(i * _L, _L)] = zf
        return 0

    lax.fori_loop(0, H * _NPAD // _L, zbody, 0)

    ms = []
    for h in range(H):
        mv = jnp.maximum(mxp_v[0, h, :], mxp_v[1, h, :])
        ms.append(jnp.max(mv))

    def body(i, _):
        b = i * _L
        dv = dst_v[pl.ds(b, _L)]
        for h in range(H):
            a = jnp.exp(score_v[pl.ds(h * _EPW + b, _L)] - ms[h])
            score_v[pl.ds(h * _EPW + b, _L)] = a
            plsc.addupdate_scatter(den_v, [dv + h * _NPAD], a)
        return 0

    lax.fori_loop(0, _NIT, body, 0)
    pltpu.sync_copy(score_v, att_hbm.at[w])
    pltpu.sync_copy(den_v, denpart_hbm.at[w])


def _att_b(H, score, dst, maxpart):
    k = pl.kernel(
        functools.partial(_att_b_body, H),
        out_type=[
            jax.ShapeDtypeStruct((_NW, H * _EPW), jnp.float32),
            jax.ShapeDtypeStruct((_NW, H * _NPAD), jnp.float32),
        ],
        mesh=_mesh,
        compiler_params=pltpu.CompilerParams(needs_layout_passes=False),
        scratch_types=[
            pltpu.VMEM((H * _EPW,), jnp.float32),
            pltpu.VMEM((_EPW,), jnp.int32),
            pltpu.VMEM((_NC, H, _L), jnp.float32),
            pltpu.VMEM((H * _NPAD,), jnp.float32),
        ],
    )
    return k(score, dst, maxpart)


_DSTRIPE = 1280


def _den_body(H, denpart_hbm, rec_hbm, part_v, out_v):
    c = lax.axis_index("c")
    t = lax.axis_index("s")
    w = c * _NS + t
    nstripes = H * _NPAD // _DSTRIPE

    @pl.when(w < nstripes)
    def _():
        base = pl.multiple_of(w * _DSTRIPE, _DSTRIPE)
        pltpu.sync_copy(denpart_hbm.at[:, pl.ds(base, _DSTRIPE)], part_v)

        def body(j, _):
            b = j * _L
            acc = part_v[0, pl.ds(b, _L)]
            for r in range(1, _NW):
                acc = acc + part_v[r, pl.ds(b, _L)]
            out_v[pl.ds(b, _L)] = 1.0 / (acc + 1e-9)
            return 0

        lax.fori_loop(0, _DSTRIPE // _L, body, 0)
        pltpu.sync_copy(out_v, rec_hbm.at[pl.ds(base, _DSTRIPE)])


def _den(H, denpart):
    k = pl.kernel(
        functools.partial(_den_body, H),
        out_type=jax.ShapeDtypeStruct((H * _NPAD,), jnp.float32),
        mesh=_mesh,
        compiler_params=pltpu.CompilerParams(needs_layout_passes=False),
        scratch_types=[
            pltpu.VMEM((_NW, _DSTRIPE), jnp.float32),
            pltpu.VMEM((_DSTRIPE,), jnp.float32),
        ],
    )
    return k(denpart)


_BLK = 256
_SELC = 5120
_S2C = _BLK + _G
_RPT = _CH // _NS


def _acc_body(H, D, wh_hbm, att_hbm, src_hbm, dst_hbm, rec_hbm, out_hbm,
              src_v, dst_v, att_v, seli_v, self_v, blki_v, blkf_v,
              s2i_v, s2f_v, cnt_v, cntf_v, rec_v, rowbuf, acc_v,
              shared_segi, shared_segf, shared_cnt):
    c = lax.axis_index("c")
    t = lax.axis_index("s")
    w = c * _NS + t
    DH = D // H
    pltpu.sync_copy(src_hbm.at[pl.ds(w * _EPW, _EPW)], src_v)
    pltpu.sync_copy(dst_hbm.at[pl.ds(w * _EPW, _EPW)], dst_v)
    pltpu.sync_copy(att_hbm.at[w], att_v)

    zf = jnp.zeros((_L,), jnp.float32)
    zi = jnp.zeros((_L,), jnp.int32)
    iota = jax.lax.iota(jnp.int32, _L)
    rlo = t * _RPT

    def kbody(kk, _):
        lo = pl.multiple_of(kk * _CH, _CH)
        for h in range(H):
            pltpu.sync_copy(rec_hbm.at[pl.ds(pl.multiple_of(h * _NPAD + lo, _CH), _CH)],
                            rec_v.at[pl.ds(h * _CH, _CH)])

        def cbody(i, cur):
            b = i * _L
            dv = dst_v[pl.ds(b, _L)]
            sv = src_v[pl.ds(b, _L)]
            m = (dv >= lo) & (dv < lo + _CH)
            dl = dv - lo
            dlc = jnp.where(m, dl, 0)
            plsc.store_compressed(seli_v.at[pl.ds(cur, _L)], sv, mask=m)
            plsc.store_compressed(seli_v.at[pl.ds(_SELC + cur, _L)], dl, mask=m)
            for h in range(H):
                a = att_v[pl.ds(h * _EPW + b, _L)]
                r = plsc.load_gather(rec_v, [dlc + h * _CH])
                plsc.store_compressed(
                    self_v.at[pl.ds(h * _SELC + cur, _L)], a * r, mask=m)
            return cur + jnp.sum(m.astype(jnp.int32))

        count = lax.fori_loop(0, _NIT, cbody, jnp.int32(0))
        pltpu.sync_copy(
            seli_v, shared_segi.at[pl.ds(pl.multiple_of(t * 2 * _SELC, 128),
                                         2 * _SELC)])
        pltpu.sync_copy(
            self_v, shared_segf.at[pl.ds(pl.multiple_of(t * H * _SELC, 128),
                                         H * _SELC)])
        cntf_v[pl.ds(0, _L)] = zi + count
        pltpu.sync_copy(cntf_v.at[pl.ds(0, _L)],
                        shared_cnt.at[pl.ds(pl.multiple_of(t * 128, 128), _L)])
        plsc.subcore_barrier()
        pltpu.sync_copy(shared_cnt, cnt_v)

        def zb(j, _):
            for r in range(_RPT):
                acc_v[r, pl.ds(j * _L, _L)] = zf
            return 0

        lax.fori_loop(0, D // _L, zb, 0)

        def obody(o, _):
            cnt_o = plsc.load_gather(cnt_v, [zi + o * 128])[0]
            nblk = (cnt_o + _BLK - 1) // _BLK

            def blkbody(bi, _):
                bb = bi * _BLK
                segib = pl.multiple_of(o * 2 * _SELC + bb, 128)
                pltpu.sync_copy(shared_segi.at[pl.ds(segib, _BLK)],
                                blki_v.at[pl.ds(0, _BLK)])
                pltpu.sync_copy(
                    shared_segi.at[pl.ds(pl.multiple_of(segib + _SELC, 128),
                                         _BLK)],
                    blki_v.at[pl.ds(_BLK, _BLK)])
                for h in range(H):
                    pltpu.sync_copy(
                        shared_segf.at[
                            pl.ds(pl.multiple_of(o * H * _SELC + h * _SELC + bb,
                                                 128), _BLK)],
                        blkf_v.at[pl.ds(h * _BLK, _BLK)])

                def fbody(i, cur2):
                    base = i * _L
                    gpos = bb + base + iota
                    dl = blki_v[pl.ds(_BLK + base, _L)]
                    mine = (gpos < cnt_o) & (dl >= rlo) & (dl < rlo + _RPT)
                    plsc.store_compressed(s2i_v.at[pl.ds(cur2, _L)],
                                          blki_v[pl.ds(base, _L)], mask=mine)
                    plsc.store_compressed(s2i_v.at[pl.ds(_S2C + cur2, _L)],
                                          dl - rlo, mask=mine)
                    for h in range(H):
                        plsc.store_compressed(
                            s2f_v.at[pl.ds(h * _S2C + cur2, _L)],
                            blkf_v[pl.ds(h * _BLK + base, _L)], mask=mine)
                    return cur2 + jnp.sum(mine.astype(jnp.int32))

                cnt2 = lax.fori_loop(0, _BLK // _L, fbody, jnp.int32(0))
                full = iota < _L
                for s in range(_G // _L):
                    plsc.store_compressed(
                        s2i_v.at[pl.ds(cnt2 + s * _L, _L)], zi, mask=full)
                    plsc.store_compressed(
                        s2i_v.at[pl.ds(_S2C + cnt2 + s * _L, _L)], zi + _RPT,
                        mask=full)
                ntrip = (cnt2 + _G - 1) // _G

                def gbody(g, _):
                    gb = g * _G
                    pltpu.sync_copy(wh_hbm.at[s2i_v.at[pl.ds(gb, _G)]], rowbuf)
                    for s in range(_G // _L):
                        rows = s2i_v[pl.ds(_S2C + gb + s * _L, _L)]
                        avs = [s2f_v[pl.ds(h * _S2C + gb + s * _L, _L)]
                               for h in range(H)]
                        for r in range(_L):
                            rowv = zi + rows[r]

                            def abody(j, _, s=s, r=r, rowv=rowv, avs=avs):
                                jb = j * _L
                                for h in range(H):
                                    off = h * DH + jb
                                    v = (rowbuf[s * _L + r, pl.ds(off, _L)]
                                         * avs[h][r])
                                    plsc.addupdate_scatter(
                                        acc_v, [rowv, off + iota], v)
                                return 0

                            lax.fori_loop(0, DH // _L, abody, 0)
                    return 0

                lax.fori_loop(0, ntrip, gbody, 0)
                return 0

            lax.fori_loop(0, nblk, blkbody, 0)
            return 0

        lax.fori_loop(0, _NS, obody, 0)

        pltpu.sync_copy(acc_v.at[pl.ds(0, _RPT)],
                        out_hbm.at[c, pl.ds(pl.multiple_of(lo + rlo, _RPT), _RPT)])
        plsc.subcore_barrier()
        return 0

    lax.fori_loop(0, _NCHUNK, kbody, 0)


def _acc(H, D, wh, att, src, dst, rec):
    k = pl.kernel(
        functools.partial(_acc_body, H, D),
        out_type=jax.ShapeDtypeStruct((_NC, _NPAD, D), jnp.float32),
        mesh=_mesh,
        compiler_params=pltpu.CompilerParams(needs_layout_passes=False),
        scratch_types=[
            pltpu.VMEM((_EPW,), jnp.int32),
            pltpu.VMEM((_EPW,), jnp.int32),
            pltpu.VMEM((H * _EPW,), jnp.float32),
            pltpu.VMEM((2 * _SELC,), jnp.int32),
            pltpu.VMEM((H * _SELC,), jnp.float32),
            pltpu.VMEM((2 * _BLK,), jnp.int32),
            pltpu.VMEM((H * _BLK,), jnp.float32),
            pltpu.VMEM((2 * _S2C,), jnp.int32),
            pltpu.VMEM((H * _S2C,), jnp.float32),
            pltpu.VMEM((_NS * 128,), jnp.int32),
            pltpu.VMEM((_L,), jnp.int32),
            pltpu.VMEM((H * _CH,), jnp.float32),
            pltpu.VMEM((_G, D), jnp.float32),
            pltpu.VMEM((_RPT + 1, D), jnp.float32),
            pltpu.VMEM_SHARED((_NS * 2 * _SELC,), jnp.int32),
            pltpu.VMEM_SHARED((_NS * H * _SELC,), jnp.float32),
            pltpu.VMEM_SHARED((_NS * 128,), jnp.int32),
        ],
    )
    return k(wh, att, src, dst, rec)


def _pad_edges(edge_index):
    pad = jnp.full((_EPAD - _E,), _N, jnp.int32)
    src = jnp.concatenate([edge_index[0], pad])
    dst = jnp.concatenate([edge_index[1], pad])
    return src, dst


def _sd_transpose(sd, H):
    sdT = sd.T
    pad_s = jnp.full((H, _NPAD - _N), -1e30, jnp.float32)
    pad_d = jnp.zeros((H, _NPAD - _N), jnp.float32)
    return jnp.concatenate(
        [sdT, jnp.concatenate([pad_s, pad_d], axis=0)], axis=1).reshape(-1)


def _gat_edge_phase(wh, sd, H, src, dst, acc_heads=None):
    D = wh.shape[1]
    DH = D // H
    sdp = _sd_transpose(sd, H)
    score, maxpart = _att_a(H, sdp, src, dst)
    att, denpart = _att_b(H, score, dst, maxpart)
    rec = _den(H, denpart)
    whp = jnp.concatenate([wh, jnp.zeros((_WHROWS - _N, D), jnp.float32)])
    if acc_heads is None or acc_heads == H:
        out = _acc(H, D, whp, att, src, dst, rec)
        return (out[0] + out[1])[:_N]
    parts = []
    for h0 in range(0, H, acc_heads):
        part = _acc(
            acc_heads, acc_heads * DH,
            whp[:, h0 * DH:(h0 + acc_heads) * DH],
            att[:, h0 * _EPW:(h0 + acc_heads) * _EPW],
            src, dst,
            rec[h0 * _NPAD:(h0 + acc_heads) * _NPAD])
        parts.append((part[0] + part[1])[:_N])
    return jnp.concatenate(parts, axis=1)


def kernel(x, params, edge_index):
    src, dst = _pad_edges(edge_index)

    W1 = jnp.concatenate([params['W%d' % i] for i in range(_NHEADS)], axis=1)
    nhid = params['W0'].shape[1]
    sblocks, dblocks = [], []
    for i in range(_NHEADS):
        col_s = jnp.zeros((nhid, _NHEADS), jnp.float32).at[:, i].set(params['a_src%d' % i][:, 0])
        col_d = jnp.zeros((nhid, _NHEADS), jnp.float32).at[:, i].set(params['a_dst%d' % i][:, 0])
        sblocks.append(col_s)
        dblocks.append(col_d)
    A1 = jnp.concatenate(
        [jnp.concatenate([s, d], axis=1) for s, d in zip(sblocks, dblocks)], axis=0)
    wh1, sd1 = _mm_scores(x, W1, A1)
    raw1 = _gat_edge_phase(wh1, sd1, _NHEADS, src, dst, acc_heads=2)

    A2 = jnp.concatenate([params['a_src_out'], params['a_dst_out']], axis=1)
    wh2, sd2 = _mm_scores(raw1, params['W_out'], A2, elu_input=True)
    emb = _gat_edge_phase(wh2, sd2, 1, src, dst)

    gene = _mlp(emb, params['g1W'], params['g1b'], params['ln_g'], params['ln_b'],
                params['g2W'], params['g2b'])
    return (emb, gene)


# speedup vs baseline: 1.0049x; 1.0049x over previous
import functools

import jax
import jax.numpy as jnp
from jax import lax
from jax.experimental import pallas as pl
from jax.experimental.pallas import tpu as pltpu
from jax.experimental.pallas import tpu_sc as plsc

_N = 10000
_E = 160000
_NHEADS = 4
_ALPHA = 0.01

_NC = 2
_NS = 16
_NW = _NC * _NS
_L = 16
_EPW = 5008
_EPAD = _NW * _EPW
_NPAD = 10240
_COLS = _NPAD // _NS
_CH = 512
_NCHUNK = _NPAD // _CH
_WHROWS = _N + 8
_NIT = _EPW // _L
_G = 16

_mesh = plsc.VectorSubcoreMesh(core_axis_name="c", subcore_axis_name="s")


def _mm_scores_body(x_ref, w_ref, a_ref, wh_ref, s_ref, *, elu_input):
    x = x_ref[...]
    if elu_input:
        x = jnp.where(x > 0, x, jnp.exp(x) - 1.0)
    wh = jnp.dot(x, w_ref[...], preferred_element_type=jnp.float32)
    wh_ref[...] = wh
    s_ref[...] = jnp.dot(wh, a_ref[...], preferred_element_type=jnp.float32)


def _mm_scores(x, W, A, block_rows=1000, elu_input=False):
    n, k = x.shape
    m = W.shape[1]
    c = A.shape[1]
    return pl.pallas_call(
        functools.partial(_mm_scores_body, elu_input=elu_input),
        grid=(n // block_rows,),
        in_specs=[
            pl.BlockSpec((block_rows, k), lambda i: (i, 0)),
            pl.BlockSpec((k, m), lambda i: (0, 0)),
            pl.BlockSpec((m, c), lambda i: (0, 0)),
        ],
        out_specs=[
            pl.BlockSpec((block_rows, m), lambda i: (i, 0)),
            pl.BlockSpec((block_rows, c), lambda i: (i, 0)),
        ],
        out_shape=[
            jax.ShapeDtypeStruct((n, m), jnp.float32),
            jax.ShapeDtypeStruct((n, c), jnp.float32),
        ],
    )(x, W, A)


def _mlp_body(emb_ref, w1_ref, b1_ref, g_ref, b_ref, w2_ref, b2_ref, out_ref):
    g = jnp.dot(emb_ref[...], w1_ref[...], preferred_element_type=jnp.float32)
    g = g + b1_ref[...]
    g = jnp.maximum(g, 0.0)
    mu = jnp.mean(g, axis=-1, keepdims=True)
    var = jnp.mean((g - mu) ** 2, axis=-1, keepdims=True)
    g = (g - mu) / jnp.sqrt(var + 1e-5) * g_ref[...] + b_ref[...]
    out_ref[...] = jnp.dot(g, w2_ref[...], preferred_element_type=jnp.float32) + b2_ref[...]


def _mlp(emb, w1, b1, ln_g, ln_b, w2, b2, block_rows=1000):
    n, k = emb.shape
    h = w1.shape[1]
    m = w2.shape[1]
    return pl.pallas_call(
        _mlp_body,
        grid=(n // block_rows,),
        in_specs=[
            pl.BlockSpec((block_rows, k), lambda i: (i, 0)),
            pl.BlockSpec((k, h), lambda i: (0, 0)),
            pl.BlockSpec((1, h), lambda i: (0, 0)),
            pl.BlockSpec((1, h), lambda i: (0, 0)),
            pl.BlockSpec((1, h), lambda i: (0, 0)),
            pl.BlockSpec((h, m), lambda i: (0, 0)),
            pl.BlockSpec((1, m), lambda i: (0, 0)),
        ],
        out_specs=pl.BlockSpec((block_rows, m), lambda i: (i, 0)),
        out_shape=jax.ShapeDtypeStruct((n, m), jnp.float32),
    )(emb, w1, b1.reshape(1, h), ln_g.reshape(1, h),
      ln_b.reshape(1, h), w2, b2.reshape(1, m))


def _att_a_body(H, sd_hbm, src_hbm, dst_hbm, score_hbm, maxpart_hbm,
                sd_v, src_v, dst_v, score_v, mx_v, red_v, shared_mx):
    c = lax.axis_index("c")
    t = lax.axis_index("s")
    w = c * _NS + t
    pltpu.sync_copy(sd_hbm, sd_v)
    pltpu.sync_copy(src_hbm.at[pl.ds(w * _EPW, _EPW)], src_v)
    pltpu.sync_copy(dst_hbm.at[pl.ds(w * _EPW, _EPW)], dst_v)

    def body(i, carry):
        b = i * _L
        sv = src_v[pl.ds(b, _L)]
        dv = dst_v[pl.ds(b, _L)]
        out = []
        for h in range(H):
            e = (plsc.load_gather(sd_v, [sv + h * _NPAD])
                 + plsc.load_gather(sd_v, [dv + (H + h) * _NPAD]))
            e = jnp.where(e > 0, e, _ALPHA * e)
            score_v[pl.ds(h * _EPW + b, _L)] = e
            out.append(jnp.maximum(carry[h], e))
        return tuple(out)

    init = tuple(jnp.full((_L,), -1e30, jnp.float32) for _ in range(H))
    mx = lax.fori_loop(0, _NIT, body, init)
    for h in range(H):
        mx_v[h, :] = mx[h]
    pltpu.sync_copy(score_v, score_hbm.at[w])
    pltpu.sync_copy(mx_v, shared_mx.at[t])
    plsc.subcore_barrier()

    @pl.when(t == 0)
    def _():
        pltpu.sync_copy(shared_mx, red_v)
        for h in range(H):
            m = red_v[0, h, :]
            for tt in range(1, _NS):
                m = jnp.maximum(m, red_v[tt, h, :])
            mx_v[h, :] = m
        pltpu.sync_copy(mx_v, maxpart_hbm.at[c])


def _att_a(H, sd, src, dst):
    k = pl.kernel(
        functools.partial(_att_a_body, H),
        out_type=[
            jax.ShapeDtypeStruct((_NW, H * _EPW), jnp.float32),
            jax.ShapeDtypeStruct((_NC, H, _L), jnp.float32),
        ],
        mesh=_mesh,
        compiler_params=pltpu.CompilerParams(needs_layout_passes=False),
        scratch_types=[
            pltpu.VMEM((2 * H * _NPAD,), jnp.float32),
            pltpu.VMEM((_EPW,), jnp.int32),
            pltpu.VMEM((_EPW,), jnp.int32),
            pltpu.VMEM((H * _EPW,), jnp.float32),
            pltpu.VMEM((H, _L), jnp.float32),
            pltpu.VMEM((_NS, H, _L), jnp.float32),
            pltpu.VMEM_SHARED((_NS, H, _L), jnp.float32),
        ],
    )
    return k(sd, src, dst)


def _att_b_body(H, score_hbm, dst_hbm, maxpart_hbm, att_hbm, denpart_hbm,
                score_v, dst_v, mxp_v, den_v):
    c = lax.axis_index("c")
    t = lax.axis_index("s")
    w = c * _NS + t
    pltpu.sync_copy(score_hbm.at[w], score_v)
    pltpu.sync_copy(dst_hbm.at[pl.ds(w * _EPW, _EPW)], dst_v)
    pltpu.sync_copy(maxpart_hbm, mxp_v)

    zf = jnp.zeros((_L,), jnp.float32)

    def zbody(i, _):
        den_v[pl.ds(i * _L, _L)] = zf
        return 0

    lax.fori_loop(0, H * _NPAD // _L, zbody, 0)

    ms = []
    for h in range(H):
        mv = jnp.maximum(mxp_v[0, h, :], mxp_v[1, h, :])
        ms.append(jnp.max(mv))

    def body(i, _):
        b = i * _L
        dv = dst_v[pl.ds(b, _L)]
        for h in range(H):
            a = jnp.exp(score_v[pl.ds(h * _EPW + b, _L)] - ms[h])
            score_v[pl.ds(h * _EPW + b, _L)] = a
            plsc.addupdate_scatter(den_v, [dv + h * _NPAD], a)
        return 0

    lax.fori_loop(0, _NIT, body, 0)
    pltpu.sync_copy(score_v, att_hbm.at[w])
    pltpu.sync_copy(den_v, denpart_hbm.at[w])


def _att_b(H, score, dst, maxpart):
    k = pl.kernel(
        functools.partial(_att_b_body, H),
        out_type=[
            jax.ShapeDtypeStruct((_NW, H * _EPW), jnp.float32),
            jax.ShapeDtypeStruct((_NW, H * _NPAD), jnp.float32),
        ],
        mesh=_mesh,
        compiler_params=pltpu.CompilerParams(needs_layout_passes=False),
        scratch_types=[
            pltpu.VMEM((H * _EPW,), jnp.float32),
            pltpu.VMEM((_EPW,), jnp.int32),
            pltpu.VMEM((_NC, H, _L), jnp.float32),
            pltpu.VMEM((H * _NPAD,), jnp.float32),
        ],
    )
    return k(score, dst, maxpart)


_DSTRIPE = 1280


def _den_body(H, denpart_hbm, rec_hbm, part_v, out_v):
    c = lax.axis_index("c")
    t = lax.axis_index("s")
    w = c * _NS + t
    nstripes = H * _NPAD // _DSTRIPE

    @pl.when(w < nstripes)
    def _():
        base = pl.multiple_of(w * _DSTRIPE, _DSTRIPE)
        pltpu.sync_copy(denpart_hbm.at[:, pl.ds(base, _DSTRIPE)], part_v)

        def body(j, _):
            b = j * _L
            acc = part_v[0, pl.ds(b, _L)]
            for r in range(1, _NW):
                acc = acc + part_v[r, pl.ds(b, _L)]
            out_v[pl.ds(b, _L)] = 1.0 / (acc + 1e-9)
            return 0

        lax.fori_loop(0, _DSTRIPE // _L, body, 0)
        pltpu.sync_copy(out_v, rec_hbm.at[pl.ds(base, _DSTRIPE)])


def _den(H, denpart):
    k = pl.kernel(
        functools.partial(_den_body, H),
        out_type=jax.ShapeDtypeStruct((H * _NPAD,), jnp.float32),
        mesh=_mesh,
        compiler_params=pltpu.CompilerParams(needs_layout_passes=False),
        scratch_types=[
            pltpu.VMEM((_NW, _DSTRIPE), jnp.float32),
            pltpu.VMEM((_DSTRIPE,), jnp.float32),
        ],
    )
    return k(denpart)


_BLK = 256
_SELC = 5120
_S2C = _BLK + _G
_RPT = _CH // _NS


def _acc_body(H, D, wh_hbm, att_hbm, src_hbm, dst_hbm, rec_hbm, out_hbm,
              src_v, dst_v, att_v, seli_v, self_v, blki_v, blkf_v,
              s2i_v, s2f_v, cnt_v, cntf_v, rec_v, rowbuf, acc_v,
              shared_segi, shared_segf, shared_cnt, dsem):
    c = lax.axis_index("c")
    t = lax.axis_index("s")
    w = c * _NS + t
    DH = D // H
    pltpu.sync_copy(src_hbm.at[pl.ds(w * _EPW, _EPW)], src_v)
    pltpu.sync_copy(dst_hbm.at[pl.ds(w * _EPW, _EPW)], dst_v)
    pltpu.sync_copy(att_hbm.at[w], att_v)

    zf = jnp.zeros((_L,), jnp.float32)
    zi = jnp.zeros((_L,), jnp.int32)
    iota = jax.lax.iota(jnp.int32, _L)
    rlo = t * _RPT

    def kbody(kk, _):
        lo = pl.multiple_of(kk * _CH, _CH)
        for h in range(H):
            pltpu.sync_copy(rec_hbm.at[pl.ds(pl.multiple_of(h * _NPAD + lo, _CH), _CH)],
                            rec_v.at[pl.ds(h * _CH, _CH)])

        def cbody(i, cur):
            b = i * _L
            dv = dst_v[pl.ds(b, _L)]
            sv = src_v[pl.ds(b, _L)]
            m = (dv >= lo) & (dv < lo + _CH)
            dl = dv - lo
            dlc = jnp.where(m, dl, 0)
            plsc.store_compressed(seli_v.at[pl.ds(cur, _L)], sv, mask=m)
            plsc.store_compressed(seli_v.at[pl.ds(_SELC + cur, _L)], dl, mask=m)
            for h in range(H):
                a = att_v[pl.ds(h * _EPW + b, _L)]
                r = plsc.load_gather(rec_v, [dlc + h * _CH])
                plsc.store_compressed(
                    self_v.at[pl.ds(h * _SELC + cur, _L)], a * r, mask=m)
            return cur + jnp.sum(m.astype(jnp.int32))

        count = lax.fori_loop(0, _NIT, cbody, jnp.int32(0))
        pltpu.sync_copy(
            seli_v, shared_segi.at[pl.ds(pl.multiple_of(t * 2 * _SELC, 128),
                                         2 * _SELC)])
        pltpu.sync_copy(
            self_v, shared_segf.at[pl.ds(pl.multiple_of(t * H * _SELC, 128),
                                         H * _SELC)])
        cntf_v[pl.ds(0, _L)] = zi + count
        pltpu.sync_copy(cntf_v.at[pl.ds(0, _L)],
                        shared_cnt.at[pl.ds(pl.multiple_of(t * 128, 128), _L)])
        plsc.subcore_barrier()
        pltpu.sync_copy(shared_cnt, cnt_v)

        def zb(j, _):
            for r in range(_RPT):
                acc_v[r, pl.ds(j * _L, _L)] = zf
            return 0

        lax.fori_loop(0, D // _L, zb, 0)

        def obody(o, _):
            cnt_o = plsc.load_gather(cnt_v, [zi + o * 128])[0]
            nblk = (cnt_o + _BLK - 1) // _BLK

            def blkbody(bi, _):
                bb = bi * _BLK
                segib = pl.multiple_of(o * 2 * _SELC + bb, 128)
                cps = [
                    pltpu.async_copy(shared_segi.at[pl.ds(segib, _BLK)],
                                     blki_v.at[pl.ds(0, _BLK)], dsem),
                    pltpu.async_copy(
                        shared_segi.at[pl.ds(pl.multiple_of(segib + _SELC, 128),
                                             _BLK)],
                        blki_v.at[pl.ds(_BLK, _BLK)], dsem),
                ]
                for h in range(H):
                    cps.append(pltpu.async_copy(
                        shared_segf.at[
                            pl.ds(pl.multiple_of(o * H * _SELC + h * _SELC + bb,
                                                 128), _BLK)],
                        blkf_v.at[pl.ds(h * _BLK, _BLK)], dsem))
                for cp in cps:
                    cp.wait()

                def fbody(i, cur2):
                    base = i * _L
                    gpos = bb + base + iota
                    dl = blki_v[pl.ds(_BLK + base, _L)]
                    mine = (gpos < cnt_o) & (dl >= rlo) & (dl < rlo + _RPT)
                    plsc.store_compressed(s2i_v.at[pl.ds(cur2, _L)],
                                          blki_v[pl.ds(base, _L)], mask=mine)
                    plsc.store_compressed(s2i_v.at[pl.ds(_S2C + cur2, _L)],
                                          dl - rlo, mask=mine)
                    for h in range(H):
                        plsc.store_compressed(
                            s2f_v.at[pl.ds(h * _S2C + cur2, _L)],
                            blkf_v[pl.ds(h * _BLK + base, _L)], mask=mine)
                    return cur2 + jnp.sum(mine.astype(jnp.int32))

                cnt2 = lax.fori_loop(0, _BLK // _L, fbody, jnp.int32(0))
                full = iota < _L
                for s in range(_G // _L):
                    plsc.store_compressed(
                        s2i_v.at[pl.ds(cnt2 + s * _L, _L)], zi, mask=full)
                    plsc.store_compressed(
                        s2i_v.at[pl.ds(_S2C + cnt2 + s * _L, _L)], zi + _RPT,
                        mask=full)
                ntrip = (cnt2 + _G - 1) // _G

                def gbody(g, _):
                    gb = g * _G
                    pltpu.sync_copy(wh_hbm.at[s2i_v.at[pl.ds(gb, _G)]], rowbuf)
                    for s in range(_G // _L):
                        rows = s2i_v[pl.ds(_S2C + gb + s * _L, _L)]
                        avs = [s2f_v[pl.ds(h * _S2C + gb + s * _L, _L)]
                               for h in range(H)]
                        for r in range(_L):
                            rowv = zi + rows[r]

                            def abody(j, _, s=s, r=r, rowv=rowv, avs=avs):
                                jb = j * _L
                                for h in range(H):
                                    off = h * DH + jb
                                    v = (rowbuf[s * _L + r, pl.ds(off, _L)]
                                         * avs[h][r])
                                    plsc.addupdate_scatter(
                                        acc_v, [rowv, off + iota], v)
                                return 0

                            lax.fori_loop(0, DH // _L, abody, 0)
                    return 0

                lax.fori_loop(0, ntrip, gbody, 0)
                return 0

            lax.fori_loop(0, nblk, blkbody, 0)
            return 0

        lax.fori_loop(0, _NS, obody, 0)

        pltpu.sync_copy(acc_v.at[pl.ds(0, _RPT)],
                        out_hbm.at[c, pl.ds(pl.multiple_of(lo + rlo, _RPT), _RPT)])
        plsc.subcore_barrier()
        return 0

    lax.fori_loop(0, _NCHUNK, kbody, 0)


def _acc(H, D, wh, att, src, dst, rec):
    k = pl.kernel(
        functools.partial(_acc_body, H, D),
        out_type=jax.ShapeDtypeStruct((_NC, _NPAD, D), jnp.float32),
        mesh=_mesh,
        compiler_params=pltpu.CompilerParams(needs_layout_passes=False),
        scratch_types=[
            pltpu.VMEM((_EPW,), jnp.int32),
            pltpu.VMEM((_EPW,), jnp.int32),
            pltpu.VMEM((H * _EPW,), jnp.float32),
            pltpu.VMEM((2 * _SELC,), jnp.int32),
            pltpu.VMEM((H * _SELC,), jnp.float32),
            pltpu.VMEM((2 * _BLK,), jnp.int32),
            pltpu.VMEM((H * _BLK,), jnp.float32),
            pltpu.VMEM((2 * _S2C,), jnp.int32),
            pltpu.VMEM((H * _S2C,), jnp.float32),
            pltpu.VMEM((_NS * 128,), jnp.int32),
            pltpu.VMEM((_L,), jnp.int32),
            pltpu.VMEM((H * _CH,), jnp.float32),
            pltpu.VMEM((_G, D), jnp.float32),
            pltpu.VMEM((_RPT + 1, D), jnp.float32),
            pltpu.VMEM_SHARED((_NS * 2 * _SELC,), jnp.int32),
            pltpu.VMEM_SHARED((_NS * H * _SELC,), jnp.float32),
            pltpu.VMEM_SHARED((_NS * 128,), jnp.int32),
            pltpu.SemaphoreType.DMA,
        ],
    )
    return k(wh, att, src, dst, rec)


def _pad_edges(edge_index):
    pad = jnp.full((_EPAD - _E,), _N, jnp.int32)
    src = jnp.concatenate([edge_index[0], pad])
    dst = jnp.concatenate([edge_index[1], pad])
    return src, dst


def _sd_transpose(sd, H):
    sdT = sd.T
    pad_s = jnp.full((H, _NPAD - _N), -1e30, jnp.float32)
    pad_d = jnp.zeros((H, _NPAD - _N), jnp.float32)
    return jnp.concatenate(
        [sdT, jnp.concatenate([pad_s, pad_d], axis=0)], axis=1).reshape(-1)


def _gat_edge_phase(wh, sd, H, src, dst, acc_heads=None):
    D = wh.shape[1]
    DH = D // H
    sdp = _sd_transpose(sd, H)
    score, maxpart = _att_a(H, sdp, src, dst)
    att, denpart = _att_b(H, score, dst, maxpart)
    rec = _den(H, denpart)
    whp = jnp.concatenate([wh, jnp.zeros((_WHROWS - _N, D), jnp.float32)])
    if acc_heads is None or acc_heads == H:
        out = _acc(H, D, whp, att, src, dst, rec)
        return (out[0] + out[1])[:_N]
    parts = []
    for h0 in range(0, H, acc_heads):
        part = _acc(
            acc_heads, acc_heads * DH,
            whp[:, h0 * DH:(h0 + acc_heads) * DH],
            att[:, h0 * _EPW:(h0 + acc_heads) * _EPW],
            src, dst,
            rec[h0 * _NPAD:(h0 + acc_heads) * _NPAD])
        parts.append((part[0] + part[1])[:_N])
    return jnp.concatenate(parts, axis=1)


def kernel(x, params, edge_index):
    src, dst = _pad_edges(edge_index)

    W1 = jnp.concatenate([params['W%d' % i] for i in range(_NHEADS)], axis=1)
    nhid = params['W0'].shape[1]
    sblocks, dblocks = [], []
    for i in range(_NHEADS):
        col_s = jnp.zeros((nhid, _NHEADS), jnp.float32).at[:, i].set(params['a_src%d' % i][:, 0])
        col_d = jnp.zeros((nhid, _NHEADS), jnp.float32).at[:, i].set(params['a_dst%d' % i][:, 0])
        sblocks.append(col_s)
        dblocks.append(col_d)
    A1 = jnp.concatenate(
        [jnp.concatenate([s, d], axis=1) for s, d in zip(sblocks, dblocks)], axis=0)
    wh1, sd1 = _mm_scores(x, W1, A1)
    raw1 = _gat_edge_phase(wh1, sd1, _NHEADS, src, dst, acc_heads=2)

    A2 = jnp.concatenate([params['a_src_out'], params['a_dst_out']], axis=1)
    wh2, sd2 = _mm_scores(raw1, params['W_out'], A2, elu_input=True)
    emb = _gat_edge_phase(wh2, sd2, 1, src, dst)

    gene = _mlp(emb, params['g1W'], params['g1b'], params['ln_g'], params['ln_b'],
                params['g2W'], params['g2b'])
    return (emb, gene)


# speedup vs baseline: 1.0185x; 1.0135x over previous
import functools

import jax
import jax.numpy as jnp
from jax import lax
from jax.experimental import pallas as pl
from jax.experimental.pallas import tpu as pltpu
from jax.experimental.pallas import tpu_sc as plsc

_N = 10000
_E = 160000
_NHEADS = 4
_ALPHA = 0.01

_NC = 2
_NS = 16
_NW = _NC * _NS
_L = 16
_EPW = 5008
_EPAD = _NW * _EPW
_NPAD = 10240
_COLS = _NPAD // _NS
_CH = 512
_NCHUNK = _NPAD // _CH
_WHROWS = _N + 8
_NIT = _EPW // _L
_G = 16

_mesh = plsc.VectorSubcoreMesh(core_axis_name="c", subcore_axis_name="s")


def _mm_scores_body(x_ref, w_ref, a_ref, wh_ref, s_ref, *, elu_input):
    x = x_ref[...]
    if elu_input:
        x = jnp.where(x > 0, x, jnp.exp(x) - 1.0)
    wh = jnp.dot(x, w_ref[...], preferred_element_type=jnp.float32)
    wh_ref[...] = wh
    s_ref[...] = jnp.dot(wh, a_ref[...], preferred_element_type=jnp.float32)


def _mm_scores(x, W, A, block_rows=1000, elu_input=False):
    n, k = x.shape
    m = W.shape[1]
    c = A.shape[1]
    return pl.pallas_call(
        functools.partial(_mm_scores_body, elu_input=elu_input),
        grid=(n // block_rows,),
        in_specs=[
            pl.BlockSpec((block_rows, k), lambda i: (i, 0)),
            pl.BlockSpec((k, m), lambda i: (0, 0)),
            pl.BlockSpec((m, c), lambda i: (0, 0)),
        ],
        out_specs=[
            pl.BlockSpec((block_rows, m), lambda i: (i, 0)),
            pl.BlockSpec((block_rows, c), lambda i: (i, 0)),
        ],
        out_shape=[
            jax.ShapeDtypeStruct((n, m), jnp.float32),
            jax.ShapeDtypeStruct((n, c), jnp.float32),
        ],
    )(x, W, A)


def _mlp_body(emb_ref, w1_ref, b1_ref, g_ref, b_ref, w2_ref, b2_ref, out_ref):
    g = jnp.dot(emb_ref[...], w1_ref[...], preferred_element_type=jnp.float32)
    g = g + b1_ref[...]
    g = jnp.maximum(g, 0.0)
    mu = jnp.mean(g, axis=-1, keepdims=True)
    var = jnp.mean((g - mu) ** 2, axis=-1, keepdims=True)
    g = (g - mu) / jnp.sqrt(var + 1e-5) * g_ref[...] + b_ref[...]
    out_ref[...] = jnp.dot(g, w2_ref[...], preferred_element_type=jnp.float32) + b2_ref[...]


def _mlp(emb, w1, b1, ln_g, ln_b, w2, b2, block_rows=1000):
    n, k = emb.shape
    h = w1.shape[1]
    m = w2.shape[1]
    return pl.pallas_call(
        _mlp_body,
        grid=(n // block_rows,),
        in_specs=[
            pl.BlockSpec((block_rows, k), lambda i: (i, 0)),
            pl.BlockSpec((k, h), lambda i: (0, 0)),
            pl.BlockSpec((1, h), lambda i: (0, 0)),
            pl.BlockSpec((1, h), lambda i: (0, 0)),
            pl.BlockSpec((1, h), lambda i: (0, 0)),
            pl.BlockSpec((h, m), lambda i: (0, 0)),
            pl.BlockSpec((1, m), lambda i: (0, 0)),
        ],
        out_specs=pl.BlockSpec((block_rows, m), lambda i: (i, 0)),
        out_shape=jax.ShapeDtypeStruct((n, m), jnp.float32),
    )(emb, w1, b1.reshape(1, h), ln_g.reshape(1, h),
      ln_b.reshape(1, h), w2, b2.reshape(1, m))


def _att_a_body(H, sd_hbm, src_hbm, dst_hbm, score_hbm, maxpart_hbm,
                sd_v, src_v, dst_v, score_v, mx_v, red_v, shared_mx):
    c = lax.axis_index("c")
    t = lax.axis_index("s")
    w = c * _NS + t
    pltpu.sync_copy(sd_hbm, sd_v)
    pltpu.sync_copy(src_hbm.at[pl.ds(w * _EPW, _EPW)], src_v)
    pltpu.sync_copy(dst_hbm.at[pl.ds(w * _EPW, _EPW)], dst_v)

    def body(i, carry):
        b = i * _L
        sv = src_v[pl.ds(b, _L)]
        dv = dst_v[pl.ds(b, _L)]
        out = []
        for h in range(H):
            e = (plsc.load_gather(sd_v, [sv + h * _NPAD])
                 + plsc.load_gather(sd_v, [dv + (H + h) * _NPAD]))
            e = jnp.where(e > 0, e, _ALPHA * e)
            score_v[pl.ds(h * _EPW + b, _L)] = e
            out.append(jnp.maximum(carry[h], e))
        return tuple(out)

    init = tuple(jnp.full((_L,), -1e30, jnp.float32) for _ in range(H))
    mx = lax.fori_loop(0, _NIT, body, init)
    for h in range(H):
        mx_v[h, :] = mx[h]
    pltpu.sync_copy(score_v, score_hbm.at[w])
    pltpu.sync_copy(mx_v, shared_mx.at[t])
    plsc.subcore_barrier()

    @pl.when(t == 0)
    def _():
        pltpu.sync_copy(shared_mx, red_v)
        for h in range(H):
            m = red_v[0, h, :]
            for tt in range(1, _NS):
                m = jnp.maximum(m, red_v[tt, h, :])
            mx_v[h, :] = m
        pltpu.sync_copy(mx_v, maxpart_hbm.at[c])


def _att_a(H, sd, src, dst):
    k = pl.kernel(
        functools.partial(_att_a_body, H),
        out_type=[
            jax.ShapeDtypeStruct((_NW, H * _EPW), jnp.float32),
            jax.ShapeDtypeStruct((_NC, H, _L), jnp.float32),
        ],
        mesh=_mesh,
        compiler_params=pltpu.CompilerParams(needs_layout_passes=False),
        scratch_types=[
            pltpu.VMEM((2 * H * _NPAD,), jnp.float32),
            pltpu.VMEM((_EPW,), jnp.int32),
            pltpu.VMEM((_EPW,), jnp.int32),
            pltpu.VMEM((H * _EPW,), jnp.float32),
            pltpu.VMEM((H, _L), jnp.float32),
            pltpu.VMEM((_NS, H, _L), jnp.float32),
            pltpu.VMEM_SHARED((_NS, H, _L), jnp.float32),
        ],
    )
    return k(sd, src, dst)


def _att_b_body(H, score_hbm, dst_hbm, maxpart_hbm, att_hbm, denpart_hbm,
                score_v, dst_v, mxp_v, den_v):
    c = lax.axis_index("c")
    t = lax.axis_index("s")
    w = c * _NS + t
    pltpu.sync_copy(score_hbm.at[w], score_v)
    pltpu.sync_copy(dst_hbm.at[pl.ds(w * _EPW, _EPW)], dst_v)
    pltpu.sync_copy(maxpart_hbm, mxp_v)

    zf = jnp.zeros((_L,), jnp.float32)

    def zbody(i, _):
        den_v[pl.ds(i * _L, _L)] = zf
        return 0

    lax.fori_loop(0, H * _NPAD // _L, zbody, 0)

    ms = []
    for h in range(H):
        mv = jnp.maximum(mxp_v[0, h, :], mxp_v[1, h, :])
        ms.append(jnp.max(mv))

    def body(i, _):
        b = i * _L
        dv = dst_v[pl.ds(b, _L)]
        for h in range(H):
            a = jnp.exp(score_v[pl.ds(h * _EPW + b, _L)] - ms[h])
            score_v[pl.ds(h * _EPW + b, _L)] = a
            plsc.addupdate_scatter(den_v, [dv + h * _NPAD], a)
        return 0

    lax.fori_loop(0, _NIT, body, 0)
    pltpu.sync_copy(score_v, att_hbm.at[w])
    pltpu.sync_copy(den_v, denpart_hbm.at[w])


def _att_b(H, score, dst, maxpart):
    k = pl.kernel(
        functools.partial(_att_b_body, H),
        out_type=[
            jax.ShapeDtypeStruct((_NW, H * _EPW), jnp.float32),
            jax.ShapeDtypeStruct((_NW, H * _NPAD), jnp.float32),
        ],
        mesh=_mesh,
        compiler_params=pltpu.CompilerParams(needs_layout_passes=False),
        scratch_types=[
            pltpu.VMEM((H * _EPW,), jnp.float32),
            pltpu.VMEM((_EPW,), jnp.int32),
            pltpu.VMEM((_NC, H, _L), jnp.float32),
            pltpu.VMEM((H * _NPAD,), jnp.float32),
        ],
    )
    return k(score, dst, maxpart)


_DSTRIPE = 1280


def _den_body(H, denpart_hbm, rec_hbm, part_v, out_v):
    c = lax.axis_index("c")
    t = lax.axis_index("s")
    w = c * _NS + t
    nstripes = H * _NPAD // _DSTRIPE

    @pl.when(w < nstripes)
    def _():
        base = pl.multiple_of(w * _DSTRIPE, _DSTRIPE)
        pltpu.sync_copy(denpart_hbm.at[:, pl.ds(base, _DSTRIPE)], part_v)

        def body(j, _):
            b = j * _L
            acc = part_v[0, pl.ds(b, _L)]
            for r in range(1, _NW):
                acc = acc + part_v[r, pl.ds(b, _L)]
            out_v[pl.ds(b, _L)] = 1.0 / (acc + 1e-9)
            return 0

        lax.fori_loop(0, _DSTRIPE // _L, body, 0)
        pltpu.sync_copy(out_v, rec_hbm.at[pl.ds(base, _DSTRIPE)])


def _den(H, denpart):
    k = pl.kernel(
        functools.partial(_den_body, H),
        out_type=jax.ShapeDtypeStruct((H * _NPAD,), jnp.float32),
        mesh=_mesh,
        compiler_params=pltpu.CompilerParams(needs_layout_passes=False),
        scratch_types=[
            pltpu.VMEM((_NW, _DSTRIPE), jnp.float32),
            pltpu.VMEM((_DSTRIPE,), jnp.float32),
        ],
    )
    return k(denpart)


_BLK = 256
_SELC = 5120
_S2C = _BLK + _G
_RPT = _CH // _NS


def _acc_body(H, D, wh_hbm, att_hbm, src_hbm, dst_hbm, rec_hbm, out_hbm,
              src_v, dst_v, att_v, seli_v, self_v, blki_v, blkf_v,
              s2i_v, s2f_v, cnt_v, cntf_v, rec_v, rowbuf, acc_v,
              shared_segi, shared_segf, shared_cnt, dsem):
    c = lax.axis_index("c")
    t = lax.axis_index("s")
    w = c * _NS + t
    DH = D // H
    pltpu.sync_copy(src_hbm.at[pl.ds(w * _EPW, _EPW)], src_v)
    pltpu.sync_copy(dst_hbm.at[pl.ds(w * _EPW, _EPW)], dst_v)
    pltpu.sync_copy(att_hbm.at[w], att_v)

    zf = jnp.zeros((_L,), jnp.float32)
    zi = jnp.zeros((_L,), jnp.int32)
    iota = jax.lax.iota(jnp.int32, _L)
    rlo = t * _RPT

    def kbody(kk, _):
        lo = pl.multiple_of(kk * _CH, _CH)
        for h in range(H):
            pltpu.sync_copy(rec_hbm.at[pl.ds(pl.multiple_of(h * _NPAD + lo, _CH), _CH)],
                            rec_v.at[pl.ds(h * _CH, _CH)])

        def cbody(i, cur):
            b = i * _L
            dv = dst_v[pl.ds(b, _L)]
            sv = src_v[pl.ds(b, _L)]
            m = (dv >= lo) & (dv < lo + _CH)
            dl = dv - lo
            dlc = jnp.where(m, dl, 0)
            plsc.store_compressed(seli_v.at[pl.ds(cur, _L)], sv, mask=m)
            plsc.store_compressed(seli_v.at[pl.ds(_SELC + cur, _L)], dl, mask=m)
            for h in range(H):
                a = att_v[pl.ds(h * _EPW + b, _L)]
                r = plsc.load_gather(rec_v, [dlc + h * _CH])
                plsc.store_compressed(
                    self_v.at[pl.ds(h * _SELC + cur, _L)], a * r, mask=m)
            return cur + jnp.sum(m.astype(jnp.int32))

        count = lax.fori_loop(0, _NIT, cbody, jnp.int32(0))
        pltpu.sync_copy(
            seli_v, shared_segi.at[pl.ds(pl.multiple_of(t * 2 * _SELC, 128),
                                         2 * _SELC)])
        pltpu.sync_copy(
            self_v, shared_segf.at[pl.ds(pl.multiple_of(t * H * _SELC, 128),
                                         H * _SELC)])
        cntf_v[pl.ds(0, _L)] = zi + count
        pltpu.sync_copy(cntf_v.at[pl.ds(0, _L)],
                        shared_cnt.at[pl.ds(pl.multiple_of(t * 128, 128), _L)])
        plsc.subcore_barrier()
        pltpu.sync_copy(shared_cnt, cnt_v)

        def zb(j, _):
            for r in range(_RPT):
                acc_v[r, pl.ds(j * _L, _L)] = zf
            return 0

        lax.fori_loop(0, D // _L, zb, 0)

        def obody(o, _):
            cnt_o = plsc.load_gather(cnt_v, [zi + o * 128])[0]
            nblk = (cnt_o + _BLK - 1) // _BLK

            def blkbody(bi, _):
                bb = bi * _BLK
                segib = pl.multiple_of(o * 2 * _SELC + bb, 128)
                cps = [
                    pltpu.async_copy(shared_segi.at[pl.ds(segib, _BLK)],
                                     blki_v.at[pl.ds(0, _BLK)], dsem),
                    pltpu.async_copy(
                        shared_segi.at[pl.ds(pl.multiple_of(segib + _SELC, 128),
                                             _BLK)],
                        blki_v.at[pl.ds(_BLK, _BLK)], dsem),
                ]
                for h in range(H):
                    cps.append(pltpu.async_copy(
                        shared_segf.at[
                            pl.ds(pl.multiple_of(o * H * _SELC + h * _SELC + bb,
                                                 128), _BLK)],
                        blkf_v.at[pl.ds(h * _BLK, _BLK)], dsem))
                for cp in cps:
                    cp.wait()

                def fbody(i, cur2):
                    base = i * _L
                    gpos = bb + base + iota
                    dl = blki_v[pl.ds(_BLK + base, _L)]
                    mine = (gpos < cnt_o) & (dl >= rlo) & (dl < rlo + _RPT)
                    plsc.store_compressed(s2i_v.at[pl.ds(cur2, _L)],
                                          blki_v[pl.ds(base, _L)], mask=mine)
                    plsc.store_compressed(s2i_v.at[pl.ds(_S2C + cur2, _L)],
                                          dl - rlo, mask=mine)
                    for h in range(H):
                        plsc.store_compressed(
                            s2f_v.at[pl.ds(h * _S2C + cur2, _L)],
                            blkf_v[pl.ds(h * _BLK + base, _L)], mask=mine)
                    return cur2 + jnp.sum(mine.astype(jnp.int32))

                cnt2 = lax.fori_loop(0, _BLK // _L, fbody, jnp.int32(0))
                full = iota < _L
                for s in range(_G // _L):
                    plsc.store_compressed(
                        s2i_v.at[pl.ds(cnt2 + s * _L, _L)], zi, mask=full)
                    plsc.store_compressed(
                        s2i_v.at[pl.ds(_S2C + cnt2 + s * _L, _L)], zi + _RPT,
                        mask=full)
                ntrip = (cnt2 + _G - 1) // _G

                def gbody(g, _):
                    gb = g * _G
                    pltpu.sync_copy(wh_hbm.at[s2i_v.at[pl.ds(gb, _G)]], rowbuf)
                    rowvs, avls = [], []
                    for s in range(_G // _L):
                        rows = s2i_v[pl.ds(_S2C + gb + s * _L, _L)]
                        avs = [s2f_v[pl.ds(h * _S2C + gb + s * _L, _L)]
                               for h in range(H)]
                        for r in range(_L):
                            rowvs.append(zi + rows[r])
                            avls.append([avs[h][r] for h in range(H)])

                    def abody(j, _):
                        jb = j * _L
                        for idx in range(_G):
                            for h in range(H):
                                off = h * DH + jb
                                v = (rowbuf[idx, pl.ds(off, _L)]
                                     * avls[idx][h])
                                plsc.addupdate_scatter(
                                    acc_v, [rowvs[idx], off + iota], v)
                        return 0

                    lax.fori_loop(0, DH // _L, abody, 0)
                    return 0

                lax.fori_loop(0, ntrip, gbody, 0)
                return 0

            lax.fori_loop(0, nblk, blkbody, 0)
            return 0

        lax.fori_loop(0, _NS, obody, 0)

        pltpu.sync_copy(acc_v.at[pl.ds(0, _RPT)],
                        out_hbm.at[c, pl.ds(pl.multiple_of(lo + rlo, _RPT), _RPT)])
        plsc.subcore_barrier()
        return 0

    lax.fori_loop(0, _NCHUNK, kbody, 0)


def _acc(H, D, wh, att, src, dst, rec):
    k = pl.kernel(
        functools.partial(_acc_body, H, D),
        out_type=jax.ShapeDtypeStruct((_NC, _NPAD, D), jnp.float32),
        mesh=_mesh,
        compiler_params=pltpu.CompilerParams(needs_layout_passes=False),
        scratch_types=[
            pltpu.VMEM((_EPW,), jnp.int32),
            pltpu.VMEM((_EPW,), jnp.int32),
            pltpu.VMEM((H * _EPW,), jnp.float32),
            pltpu.VMEM((2 * _SELC,), jnp.int32),
            pltpu.VMEM((H * _SELC,), jnp.float32),
            pltpu.VMEM((2 * _BLK,), jnp.int32),
            pltpu.VMEM((H * _BLK,), jnp.float32),
            pltpu.VMEM((2 * _S2C,), jnp.int32),
            pltpu.VMEM((H * _S2C,), jnp.float32),
            pltpu.VMEM((_NS * 128,), jnp.int32),
            pltpu.VMEM((_L,), jnp.int32),
            pltpu.VMEM((H * _CH,), jnp.float32),
            pltpu.VMEM((_G, D), jnp.float32),
            pltpu.VMEM((_RPT + 1, D), jnp.float32),
            pltpu.VMEM_SHARED((_NS * 2 * _SELC,), jnp.int32),
            pltpu.VMEM_SHARED((_NS * H * _SELC,), jnp.float32),
            pltpu.VMEM_SHARED((_NS * 128,), jnp.int32),
            pltpu.SemaphoreType.DMA,
        ],
    )
    return k(wh, att, src, dst, rec)


def _pad_edges(edge_index):
    pad = jnp.full((_EPAD - _E,), _N, jnp.int32)
    src = jnp.concatenate([edge_index[0], pad])
    dst = jnp.concatenate([edge_index[1], pad])
    return src, dst


def _sd_transpose(sd, H):
    sdT = sd.T
    pad_s = jnp.full((H, _NPAD - _N), -1e30, jnp.float32)
    pad_d = jnp.zeros((H, _NPAD - _N), jnp.float32)
    return jnp.concatenate(
        [sdT, jnp.concatenate([pad_s, pad_d], axis=0)], axis=1).reshape(-1)


def _gat_edge_phase(wh, sd, H, src, dst, acc_heads=None):
    D = wh.shape[1]
    DH = D // H
    sdp = _sd_transpose(sd, H)
    score, maxpart = _att_a(H, sdp, src, dst)
    att, denpart = _att_b(H, score, dst, maxpart)
    rec = _den(H, denpart)
    whp = jnp.concatenate([wh, jnp.zeros((_WHROWS - _N, D), jnp.float32)])
    if acc_heads is None or acc_heads == H:
        out = _acc(H, D, whp, att, src, dst, rec)
        return (out[0] + out[1])[:_N]
    parts = []
    for h0 in range(0, H, acc_heads):
        part = _acc(
            acc_heads, acc_heads * DH,
            whp[:, h0 * DH:(h0 + acc_heads) * DH],
            att[:, h0 * _EPW:(h0 + acc_heads) * _EPW],
            src, dst,
            rec[h0 * _NPAD:(h0 + acc_heads) * _NPAD])
        parts.append((part[0] + part[1])[:_N])
    return jnp.concatenate(parts, axis=1)


def kernel(x, params, edge_index):
    src, dst = _pad_edges(edge_index)

    W1 = jnp.concatenate([params['W%d' % i] for i in range(_NHEADS)], axis=1)
    nhid = params['W0'].shape[1]
    sblocks, dblocks = [], []
    for i in range(_NHEADS):
        col_s = jnp.zeros((nhid, _NHEADS), jnp.float32).at[:, i].set(params['a_src%d' % i][:, 0])
        col_d = jnp.zeros((nhid, _NHEADS), jnp.float32).at[:, i].set(params['a_dst%d' % i][:, 0])
        sblocks.append(col_s)
        dblocks.append(col_d)
    A1 = jnp.concatenate(
        [jnp.concatenate([s, d], axis=1) for s, d in zip(sblocks, dblocks)], axis=0)
    wh1, sd1 = _mm_scores(x, W1, A1)
    raw1 = _gat_edge_phase(wh1, sd1, _NHEADS, src, dst, acc_heads=2)

    A2 = jnp.concatenate([params['a_src_out'], params['a_dst_out']], axis=1)
    wh2, sd2 = _mm_scores(raw1, params['W_out'], A2, elu_input=True)
    emb = _gat_edge_phase(wh2, sd2, 1, src, dst)

    gene = _mlp(emb, params['g1W'], params['g1b'], params['ln_g'], params['ln_b'],
                params['g2W'], params['g2b'])
    return (emb, gene)
